# baseline (speedup 1.0000x reference)
import functools
import math

import jax
import jax.numpy as jnp
from jax import lax
from jax.experimental import pallas as pl
from jax.experimental.pallas import tpu as pltpu
from jax.experimental.pallas import tpu_sc as plsc

CUTOFF = 5.0
N_RBF = 20


def _mlp_body(x_ref, w1_ref, b1_ref, w2_ref, b2_ref, o_ref):
    x = x_ref[...]
    h = jnp.dot(x, w1_ref[...], preferred_element_type=jnp.float32,
                precision=lax.Precision.HIGHEST) + b1_ref[...]
    h = h * jax.nn.sigmoid(h)
    inv = jnp.dot(h, w2_ref[...], preferred_element_type=jnp.float32,
                  precision=lax.Precision.HIGHEST) + b2_ref[...]
    half = inv.shape[1] // 2
    lo = inv[:, :half].astype(jnp.bfloat16).astype(jnp.float32)
    hi = inv[:, half:].astype(jnp.bfloat16).astype(jnp.float32)
    lo_bits = lax.shift_right_logical(
        lax.bitcast_convert_type(lo, jnp.int32), 16)
    hi_bits = lax.bitcast_convert_type(hi, jnp.int32) & jnp.int32(-65536)
    o_ref[...] = lo_bits | hi_bits


def _mlp(s_j, W1, b1, W2, b2):
    n, d_in = s_j.shape
    d_out = W2.shape[1]
    bn = 2000
    grid = (n // bn,)
    return pl.pallas_call(
        _mlp_body,
        grid=grid,
        in_specs=[
            pl.BlockSpec((bn, d_in), lambda i: (i, 0)),
            pl.BlockSpec((d_in, d_in), lambda i: (0, 0)),
            pl.BlockSpec((1, d_in), lambda i: (0, 0)),
            pl.BlockSpec((d_in, d_out), lambda i: (0, 0)),
            pl.BlockSpec((1, d_out), lambda i: (0, 0)),
        ],
        out_specs=pl.BlockSpec((bn, d_out // 2), lambda i: (i, 0)),
        out_shape=jax.ShapeDtypeStruct((n, d_out // 2), jnp.int32),
    )(s_j, W1, b1.reshape(1, d_in), W2, b2.reshape(1, d_out))


_NC = 2
_NS = 16
_NW = _NC * _NS


def _sc_gather(inv, idx):
    n, d = inv.shape
    e = idx.shape[0]
    per_w = e // _NW
    c_sub = 80
    n_sub = 5
    c = c_sub * n_sub
    n_pos = per_w // c

    mesh = plsc.VectorSubcoreMesh(core_axis_name="c", subcore_axis_name="s")

    @functools.partial(
        pl.kernel,
        out_type=jax.ShapeDtypeStruct((e, d), jnp.int32),
        mesh=mesh,
        compiler_params=pltpu.CompilerParams(use_tc_tiling_on_sc=False),
        scratch_types=[
            pltpu.VMEM((per_w,), jnp.int32),
            pltpu.VMEM((2, c, d), jnp.int32),
            pltpu.SemaphoreType.DMA,
            pltpu.SemaphoreType.DMA,
            pltpu.SemaphoreType.DMA,
            pltpu.SemaphoreType.DMA,
        ],
    )
    def sc_kernel(inv_hbm, idx_hbm, out_hbm, idx_v, rows_v,
                  gsem0, gsem1, ssem0, ssem1):
        cid = lax.axis_index("c")
        sid = lax.axis_index("s")
        wid = sid * _NC + cid
        base = wid * per_w

        pltpu.sync_copy(idx_hbm.at[pl.ds(base, per_w)], idx_v)


        gsems = (gsem0, gsem1)
        ssems = (ssem0, ssem1)

        def issue_gathers(pos, b):
            for j in range(n_sub):
                off = pos * c + j * c_sub
                pltpu.async_copy(
                    inv_hbm.at[idx_v.at[pl.ds(off, c_sub)]],
                    rows_v.at[b].at[pl.ds(j * c_sub, c_sub)],
                    gsems[b],
                )

        def wait_gathers(pos, b):
            for j in range(n_sub):
                off = pos * c + j * c_sub
                pltpu.make_async_copy(
                    inv_hbm.at[idx_v.at[pl.ds(off, c_sub)]],
                    rows_v.at[b].at[pl.ds(j * c_sub, c_sub)],
                    gsems[b],
                ).wait()

        def start_store(pos, b):
            pltpu.async_copy(
                rows_v.at[b], out_hbm.at[pl.ds(base + pos * c, c)], ssems[b])

        def wait_store(pos, b):
            pltpu.make_async_copy(
                rows_v.at[b], out_hbm.at[pl.ds(base + pos * c, c)], ssems[b]
            ).wait()

        issue_gathers(0, 0)
        issue_gathers(1, 1)

        @pl.loop(0, n_pos - 1, step=2)
        def _(p0):
            for b in range(2):
                p = p0 + b
                wait_gathers(p, b)
                start_store(p, b)
                wait_store(p, b)

                @pl.when(p + 2 < n_pos)
                def _():
                    issue_gathers(p + 2, b)

        wait_gathers(n_pos - 1, 0)
        start_store(n_pos - 1, 0)
        wait_store(n_pos - 1, 0)

    return sc_kernel(inv, idx)


_K_AUG = 24


def _wmul_body(d_ref, phi_ref, wr_ref, o_ref):
    dl = d_ref[...]
    be = dl.shape[1]
    theta = dl * (math.pi / CUTOFF)
    env = jnp.where(dl <= CUTOFF, 0.5 * (jnp.cos(theta) + 1.0), 0.0)
    denom = jnp.where(dl == 0.0, 1.0, dl)
    g = jnp.where(dl < CUTOFF, env / denom, 0.0)
    krow = lax.broadcasted_iota(jnp.int32, (_K_AUG, be), 0)
    x = (krow + 1).astype(jnp.float32) * theta
    s = jnp.sin(x) * g
    s_aug = jnp.where(krow < N_RBF, s, env)
    w = lax.dot_general(s_aug.astype(jnp.bfloat16), wr_ref[...],
                        (((0,), (0,)), ((), ())),
                        preferred_element_type=jnp.float32)
    p32 = phi_ref[...]
    half = p32.shape[1]
    lo = lax.bitcast_convert_type(lax.shift_left(p32, 16), jnp.float32)
    hi = lax.bitcast_convert_type(p32 & jnp.int32(-65536), jnp.float32)
    o_ref[:, :half] = lo * w[:, :half]
    o_ref[:, half:] = hi * w[:, half:]


def _wmul_chunk_body(d_ref, phi_ref, wr_ref, prev_ref, o_ref):
    del prev_ref
    _wmul_body(d_ref, phi_ref, wr_ref, o_ref)


def _wmul_chunk(prev_out, phi_t, dist1, wr_aug, t, e, be, nb_chunk):
    half = phi_t.shape[1]
    d_out = half * 2
    in_specs = [
        pl.BlockSpec((1, be), lambda i: (0, i + t * nb_chunk)),
        pl.BlockSpec((be, half), lambda i: (i, 0)),
        pl.BlockSpec((_K_AUG, d_out), lambda i: (0, 0)),
    ]
    args = [dist1, phi_t, wr_aug]
    aliases = {}
    body = _wmul_body
    if prev_out is not None:
        in_specs.append(pl.BlockSpec(memory_space=pl.ANY))
        args.append(prev_out)
        aliases = {3: 0}
        body = _wmul_chunk_body
    return pl.pallas_call(
        body,
        grid=(nb_chunk,),
        in_specs=in_specs,
        out_specs=pl.BlockSpec((be, d_out), lambda i: (i + t * nb_chunk, 0)),
        out_shape=jax.ShapeDtypeStruct((e, d_out), jnp.float32),
        input_output_aliases=aliases,
    )(*args)


_N_CHUNKS = 1


def kernel(s_j, dist, nbrs, W1, b1, W2, b2, Wr, br):
    e = dist.shape[0]
    d_out = W2.shape[1]
    be = 6400
    ch = e // _N_CHUNKS
    nb_chunk = ch // be
    inv = _mlp(s_j, W1, b1, W2, b2)
    idx = nbrs[:, 1]
    dist1 = dist.reshape(1, e)
    wr_aug = jnp.concatenate(
        [Wr, br.reshape(1, d_out),
         jnp.zeros((_K_AUG - N_RBF - 1, d_out), jnp.float32)], axis=0
    ).astype(jnp.bfloat16)
    phis = [_sc_gather(inv, idx[t * ch:(t + 1) * ch])
            for t in range(_N_CHUNKS)]
    out = None
    for t in range(_N_CHUNKS):
        out = _wmul_chunk(out, phis[t], dist1, wr_aug, t, e, be, nb_chunk)
    return out


# speedup vs baseline: 1.6462x; 1.6462x over previous
import functools
import math

import jax
import jax.numpy as jnp
from jax import lax
from jax.experimental import pallas as pl
from jax.experimental.pallas import tpu as pltpu
from jax.experimental.pallas import tpu_sc as plsc

CUTOFF = 5.0
N_RBF = 20


def _mlp_body(x_ref, w1_ref, b1_ref, w2_ref, b2_ref, o_ref):
    x = x_ref[...]
    h = jnp.dot(x, w1_ref[...],
                preferred_element_type=jnp.float32) + b1_ref[...]
    h = h * jax.nn.sigmoid(h)
    o_ref[...] = jnp.dot(h, w2_ref[...],
                         preferred_element_type=jnp.float32) + b2_ref[...]


def _mlp(s_j, W1, b1, W2, b2):
    n, d_in = s_j.shape
    d_out = W2.shape[1]
    bn = 2000
    grid = (n // bn,)
    return pl.pallas_call(
        _mlp_body,
        grid=grid,
        in_specs=[
            pl.BlockSpec((bn, d_in), lambda i: (i, 0)),
            pl.BlockSpec((d_in, d_in), lambda i: (0, 0)),
            pl.BlockSpec((1, d_in), lambda i: (0, 0)),
            pl.BlockSpec((d_in, d_out), lambda i: (0, 0)),
            pl.BlockSpec((1, d_out), lambda i: (0, 0)),
        ],
        out_specs=pl.BlockSpec((bn, d_out), lambda i: (i, 0)),
        out_shape=jax.ShapeDtypeStruct((n, d_out), jnp.float32),
    )(s_j, W1, b1.reshape(1, d_in), W2, b2.reshape(1, d_out))


_NC = 2
_NS = 16
_NW = _NC * _NS


def _sc_gather(inv, idx):
    n, d = inv.shape
    e = idx.shape[0]
    per_w = e // _NW
    c_sub = 80 if per_w % 400 == 0 else 40
    n_sub = 5
    c = c_sub * n_sub
    n_pos = per_w // c

    mesh = plsc.VectorSubcoreMesh(core_axis_name="c", subcore_axis_name="s")

    @functools.partial(
        pl.kernel,
        out_type=jax.ShapeDtypeStruct((e, d), jnp.float32),
        mesh=mesh,
        scratch_types=[
            pltpu.VMEM((per_w,), jnp.int32),
            pltpu.VMEM((2, c, d), jnp.float32),
            pltpu.SemaphoreType.DMA,
            pltpu.SemaphoreType.DMA,
            pltpu.SemaphoreType.DMA,
            pltpu.SemaphoreType.DMA,
        ],
    )
    def sc_kernel(inv_hbm, idx_hbm, out_hbm, idx_v, rows_v,
                  gsem0, gsem1, ssem0, ssem1):
        cid = lax.axis_index("c")
        sid = lax.axis_index("s")
        wid = sid * _NC + cid
        base = wid * per_w

        pltpu.sync_copy(idx_hbm.at[pl.ds(base, per_w)], idx_v)

        gsems = (gsem0, gsem1)
        ssems = (ssem0, ssem1)

        def issue_gathers(pos, b):
            for j in range(n_sub):
                off = pos * c + j * c_sub
                pltpu.async_copy(
                    inv_hbm.at[idx_v.at[pl.ds(off, c_sub)]],
                    rows_v.at[b].at[pl.ds(j * c_sub, c_sub)],
                    gsems[b],
                )

        def wait_gathers(pos, b):
            for j in range(n_sub):
                off = pos * c + j * c_sub
                pltpu.make_async_copy(
                    inv_hbm.at[idx_v.at[pl.ds(off, c_sub)]],
                    rows_v.at[b].at[pl.ds(j * c_sub, c_sub)],
                    gsems[b],
                ).wait()

        def start_store(pos, b):
            pltpu.async_copy(
                rows_v.at[b], out_hbm.at[pl.ds(base + pos * c, c)], ssems[b])

        def wait_store(pos, b):
            pltpu.make_async_copy(
                rows_v.at[b], out_hbm.at[pl.ds(base + pos * c, c)], ssems[b]
            ).wait()

        issue_gathers(0, 0)
        issue_gathers(1, 1)

        @pl.loop(0, n_pos - 1, step=2)
        def _(p0):
            for b in range(2):
                p = p0 + b
                wait_gathers(p, b)
                start_store(p, b)
                wait_store(p, b)

                @pl.when(p + 2 < n_pos)
                def _():
                    issue_gathers(p + 2, b)

        wait_gathers(n_pos - 1, 0)
        start_store(n_pos - 1, 0)
        wait_store(n_pos - 1, 0)

    return sc_kernel(inv, idx)


_K_AUG = 24


def _wmul_body(d_ref, phi_ref, wr_ref, o_ref):
    dl = d_ref[...]
    be = dl.shape[1]
    theta = dl * (math.pi / CUTOFF)
    env = jnp.where(dl <= CUTOFF, 0.5 * (jnp.cos(theta) + 1.0), 0.0)
    denom = jnp.where(dl == 0.0, 1.0, dl)
    g = jnp.where(dl < CUTOFF, env / denom, 0.0)
    krow = lax.broadcasted_iota(jnp.int32, (_K_AUG, be), 0)
    x = (krow + 1).astype(jnp.float32) * theta
    s = jnp.sin(x) * g
    s_aug = jnp.where(krow < N_RBF, s, env)
    w = lax.dot_general(s_aug.astype(jnp.bfloat16), wr_ref[...],
                        (((0,), (0,)), ((), ())),
                        preferred_element_type=jnp.float32)
    o_ref[...] = phi_ref[...] * w


def _wmul_chunk_body(d_ref, phi_ref, wr_ref, prev_ref, o_ref):
    del prev_ref
    _wmul_body(d_ref, phi_ref, wr_ref, o_ref)


def _wmul_chunk(prev_out, phi_t, dist1, wr_aug, t, e, be, nb_chunk):
    d_out = phi_t.shape[1]
    in_specs = [
        pl.BlockSpec((1, be), lambda i: (0, i + t * nb_chunk)),
        pl.BlockSpec((be, d_out), lambda i: (i, 0)),
        pl.BlockSpec((_K_AUG, d_out), lambda i: (0, 0)),
    ]
    args = [dist1, phi_t, wr_aug]
    aliases = {}
    body = _wmul_body
    if prev_out is not None:
        in_specs.append(pl.BlockSpec(memory_space=pl.ANY))
        args.append(prev_out)
        aliases = {3: 0}
        body = _wmul_chunk_body
    return pl.pallas_call(
        body,
        grid=(nb_chunk,),
        in_specs=in_specs,
        out_specs=pl.BlockSpec((be, d_out), lambda i: (i + t * nb_chunk, 0)),
        out_shape=jax.ShapeDtypeStruct((e, d_out), jnp.float32),
        input_output_aliases=aliases,
    )(*args)


_N_CHUNKS = 5


def kernel(s_j, dist, nbrs, W1, b1, W2, b2, Wr, br):
    e = dist.shape[0]
    d_out = W2.shape[1]
    be = 6400
    ch = e // _N_CHUNKS
    nb_chunk = ch // be
    inv = _mlp(s_j, W1, b1, W2, b2)
    idx = nbrs[:, 1]
    dist1 = dist.reshape(1, e)
    wr_aug = jnp.concatenate(
        [Wr, br.reshape(1, d_out),
         jnp.zeros((_K_AUG - N_RBF - 1, d_out), jnp.float32)], axis=0
    ).astype(jnp.bfloat16)
    phis = [_sc_gather(inv, idx[t * ch:(t + 1) * ch])
            for t in range(_N_CHUNKS)]
    out = None
    for t in range(_N_CHUNKS):
        out = _wmul_chunk(out, phis[t], dist1, wr_aug, t, e, be, nb_chunk)
    return out


# speedup vs baseline: 1.6857x; 1.0240x over previous
import functools
import math

import jax
import jax.numpy as jnp
from jax import lax
from jax.experimental import pallas as pl
from jax.experimental.pallas import tpu as pltpu
from jax.experimental.pallas import tpu_sc as plsc

CUTOFF = 5.0
N_RBF = 20


def _mlp_body(x_ref, w1_ref, b1_ref, w2_ref, b2_ref, o_ref):
    x = x_ref[...]
    h = jnp.dot(x, w1_ref[...],
                preferred_element_type=jnp.float32) + b1_ref[...]
    h = h * jax.nn.sigmoid(h)
    o_ref[...] = jnp.dot(h, w2_ref[...],
                         preferred_element_type=jnp.float32) + b2_ref[...]


def _mlp(s_j, W1, b1, W2, b2):
    n, d_in = s_j.shape
    d_out = W2.shape[1]
    bn = 2000
    grid = (n // bn,)
    return pl.pallas_call(
        _mlp_body,
        grid=grid,
        in_specs=[
            pl.BlockSpec((bn, d_in), lambda i: (i, 0)),
            pl.BlockSpec((d_in, d_in), lambda i: (0, 0)),
            pl.BlockSpec((1, d_in), lambda i: (0, 0)),
            pl.BlockSpec((d_in, d_out), lambda i: (0, 0)),
            pl.BlockSpec((1, d_out), lambda i: (0, 0)),
        ],
        out_specs=pl.BlockSpec((bn, d_out), lambda i: (i, 0)),
        out_shape=jax.ShapeDtypeStruct((n, d_out), jnp.float32),
    )(s_j, W1, b1.reshape(1, d_in), W2, b2.reshape(1, d_out))


_NC = 2
_NS = 16
_NW = _NC * _NS


def _sc_gather(inv, idx):
    n, d = inv.shape
    e = idx.shape[0]
    per_w = e // _NW
    c_sub = 80 if per_w % 400 == 0 else 40
    n_sub = 5
    c = c_sub * n_sub
    n_pos = per_w // c

    mesh = plsc.VectorSubcoreMesh(core_axis_name="c", subcore_axis_name="s")

    @functools.partial(
        pl.kernel,
        out_type=jax.ShapeDtypeStruct((e, d), jnp.float32),
        mesh=mesh,
        scratch_types=[
            pltpu.VMEM((per_w,), jnp.int32),
            pltpu.VMEM((2, c, d), jnp.float32),
            pltpu.SemaphoreType.DMA,
            pltpu.SemaphoreType.DMA,
            pltpu.SemaphoreType.DMA,
            pltpu.SemaphoreType.DMA,
        ],
    )
    def sc_kernel(inv_hbm, idx_hbm, out_hbm, idx_v, rows_v,
                  gsem0, gsem1, ssem0, ssem1):
        cid = lax.axis_index("c")
        sid = lax.axis_index("s")
        wid = sid * _NC + cid
        base = wid * per_w

        pltpu.sync_copy(idx_hbm.at[pl.ds(base, per_w)], idx_v)

        gsems = (gsem0, gsem1)
        ssems = (ssem0, ssem1)

        def issue_gathers(pos, b):
            for j in range(n_sub):
                off = pos * c + j * c_sub
                pltpu.async_copy(
                    inv_hbm.at[idx_v.at[pl.ds(off, c_sub)]],
                    rows_v.at[b].at[pl.ds(j * c_sub, c_sub)],
                    gsems[b],
                )

        def wait_gathers(pos, b):
            for j in range(n_sub):
                off = pos * c + j * c_sub
                pltpu.make_async_copy(
                    inv_hbm.at[idx_v.at[pl.ds(off, c_sub)]],
                    rows_v.at[b].at[pl.ds(j * c_sub, c_sub)],
                    gsems[b],
                ).wait()

        def start_store(pos, b):
            pltpu.async_copy(
                rows_v.at[b], out_hbm.at[pl.ds(base + pos * c, c)], ssems[b])

        def wait_store(pos, b):
            pltpu.make_async_copy(
                rows_v.at[b], out_hbm.at[pl.ds(base + pos * c, c)], ssems[b]
            ).wait()

        issue_gathers(0, 0)
        issue_gathers(1, 1)

        @pl.loop(0, n_pos - 1, step=2)
        def _(p0):
            for b in range(2):
                p = p0 + b
                wait_gathers(p, b)
                start_store(p, b)
                wait_store(p, b)

                @pl.when(p + 2 < n_pos)
                def _():
                    issue_gathers(p + 2, b)

        wait_gathers(n_pos - 1, 0)
        start_store(n_pos - 1, 0)
        wait_store(n_pos - 1, 0)

    return sc_kernel(inv, idx)


_K_AUG = 24


def _wmul_body(d_ref, phi_ref, wr_ref, o_ref):
    dl = d_ref[...]
    be = dl.shape[1]
    theta = dl * (math.pi / CUTOFF)
    env = jnp.where(dl <= CUTOFF, 0.5 * (jnp.cos(theta) + 1.0), 0.0)
    denom = jnp.where(dl == 0.0, 1.0, dl)
    g = jnp.where(dl < CUTOFF, env / denom, 0.0)
    krow = lax.broadcasted_iota(jnp.int32, (_K_AUG, be), 0)
    x = (krow + 1).astype(jnp.float32) * theta
    s = jnp.sin(x) * g
    s_aug = jnp.where(krow < N_RBF, s, env)
    w = lax.dot_general(s_aug.astype(jnp.bfloat16), wr_ref[...],
                        (((0,), (0,)), ((), ())),
                        preferred_element_type=jnp.float32)
    o_ref[...] = phi_ref[...] * w


def _wmul_chunk_body(d_ref, phi_ref, wr_ref, prev_ref, o_ref):
    del prev_ref
    _wmul_body(d_ref, phi_ref, wr_ref, o_ref)


def _wmul_chunk(prev_out, phi_t, dist1, wr_aug, t, e, be, nb_chunk):
    d_out = phi_t.shape[1]
    in_specs = [
        pl.BlockSpec((1, be), lambda i: (0, i + t * nb_chunk)),
        pl.BlockSpec((be, d_out), lambda i: (i, 0)),
        pl.BlockSpec((_K_AUG, d_out), lambda i: (0, 0)),
    ]
    args = [dist1, phi_t, wr_aug]
    aliases = {}
    body = _wmul_body
    if prev_out is not None:
        in_specs.append(pl.BlockSpec(memory_space=pl.ANY))
        args.append(prev_out)
        aliases = {3: 0}
        body = _wmul_chunk_body
    return pl.pallas_call(
        body,
        grid=(nb_chunk,),
        in_specs=in_specs,
        out_specs=pl.BlockSpec((be, d_out), lambda i: (i + t * nb_chunk, 0)),
        out_shape=jax.ShapeDtypeStruct((e, d_out), jnp.float32),
        input_output_aliases=aliases,
    )(*args)


_N_CHUNKS = 5


def kernel(s_j, dist, nbrs, W1, b1, W2, b2, Wr, br):
    e = dist.shape[0]
    d_out = W2.shape[1]
    be = 12800
    ch = e // _N_CHUNKS
    nb_chunk = ch // be
    inv = _mlp(s_j, W1, b1, W2, b2)
    idx = nbrs[:, 1]
    dist1 = dist.reshape(1, e)
    wr_aug = jnp.concatenate(
        [Wr, br.reshape(1, d_out),
         jnp.zeros((_K_AUG - N_RBF - 1, d_out), jnp.float32)], axis=0
    ).astype(jnp.bfloat16)
    phis = [_sc_gather(inv, idx[t * ch:(t + 1) * ch])
            for t in range(_N_CHUNKS)]
    out = None
    for t in range(_N_CHUNKS):
        out = _wmul_chunk(out, phis[t], dist1, wr_aug, t, e, be, nb_chunk)
    return out
